# baseline (speedup 1.0000x reference)
import functools

import jax
import jax.numpy as jnp
from jax import lax
from jax.experimental import pallas as pl
from jax.experimental.pallas import tpu as pltpu
from jax.experimental.pallas import tpu_sc as plsc

INPUT_DIM = 2048
HIDDEN_DIM = 32768
K = 64
BATCH = 32

HB = 2048
N_BLK = HIDDEN_DIM // HB

L = 16
NVREG = HIDDEN_DIM // L
CMAX_N = 512
CAP = HIDDEN_DIM + L


def _enc_body(x_ref, w_ref, b_ref, f_ref):
    acc = jax.lax.dot_general(
        x_ref[...], w_ref[...],
        (((1,), (1,)), ((), ())),
        preferred_element_type=jnp.float32,
    )
    f_ref[...] = jnp.maximum(acc + b_ref[...], 0.0) + 0.0


def _popcnt(mask):
    return plsc.all_reduce_population_count(mask)[0]


def _topk_tile(f_hbm, t_out, c_out, row_v, cmax_v, cand_v, cidx_v, stage_v):
    wid = lax.axis_index("s") * 2 + lax.axis_index("c")
    pltpu.sync_copy(f_hbm.at[wid], row_v)

    iota = lax.broadcasted_iota(jnp.int32, (L,), 0)

    def cmax_group(g, _):
        def inner(j, acc):
            return jnp.maximum(acc, row_v[pl.ds((g * 64 + j) * L, L)])
        m = lax.fori_loop(0, 64, inner, jnp.zeros((L,), jnp.int32))
        cmax_v[pl.ds(g * L, L)] = m
        return 0

    lax.fori_loop(0, 32, cmax_group, 0)

    def lb_step(i, t):
        cand = t | (jnp.int32(1) << (30 - i))
        candv = jnp.zeros((L,), jnp.int32) + cand

        def cnt_step(v, c):
            return c + _popcnt(cmax_v[pl.ds(v * L, L)] >= candv)

        cnt = lax.fori_loop(0, CMAX_N // L, cnt_step, jnp.int32(0))
        return jnp.where(cnt >= K, cand, t)

    lb = lax.fori_loop(0, 16, lb_step, jnp.int32(0))
    lbv = jnp.zeros((L,), jnp.int32) + lb

    def flt_group(g, wp):
        base = g * 8 * L
        vs = [row_v[pl.ds(base + j * L, L)] for j in range(8)]
        hit = vs[0] >= lbv
        for j in range(1, 8):
            hit = hit | (vs[j] >= lbv)
        nhit = _popcnt(hit)

        def compact(wp):
            for j in range(8):
                m = vs[j] >= lbv
                plsc.store_compressed(cand_v.at[pl.ds(wp, L)], vs[j], mask=m)
                plsc.store_compressed(
                    cidx_v.at[pl.ds(wp, L)], base + j * L + iota, mask=m)
                wp = wp + _popcnt(m)
            return wp

        return lax.cond(nhit > 0, compact, lambda w: w, wp)

    wp = lax.fori_loop(0, NVREG // 8, flt_group, jnp.int32(0))
    nv = (wp + L - 1) // L
    wpv = jnp.zeros((L,), jnp.int32) + wp

    def val_step(i, t):
        cand = t | (jnp.int32(1) << (30 - i))
        candv = jnp.zeros((L,), jnp.int32) + cand

        def cnt_step(v, c):
            valid = (v * L + iota) < wpv
            ge = (cand_v[pl.ds(v * L, L)] >= candv) & valid
            return c + _popcnt(ge)

        cnt = lax.fori_loop(0, nv, cnt_step, jnp.int32(0))
        return jnp.where(cnt >= K, cand, t)

    t = lax.fori_loop(0, 31, val_step, jnp.int32(0))
    tv = jnp.zeros((L,), jnp.int32) + t

    def gt_step(v, c):
        valid = (v * L + iota) < wpv
        gt = (cand_v[pl.ds(v * L, L)] > tv) & valid
        return c + _popcnt(gt)

    cnt_gt = lax.fori_loop(0, nv, gt_step, jnp.int32(0))
    m = K - cnt_gt

    one_v = jnp.full((L,), 1, jnp.int32)
    zero_v = jnp.zeros((L,), jnp.int32)
    neg1_v = jnp.full((L,), -1, jnp.int32)

    def tie_step(v, carry):
        cbefore, cfound = carry
        valid = (v * L + iota) < wpv
        eq = (cand_v[pl.ds(v * L, L)] == tv) & valid
        cs = plsc.cumsum(jnp.where(eq, one_v, zero_v))
        kv = jnp.zeros((L,), jnp.int32) + (m - cbefore)
        hitlane = eq & (cs == kv)
        idxv = cidx_v[pl.ds(v * L, L)]
        cnd = plsc.cummax(jnp.where(hitlane, idxv, neg1_v))[L - 1]
        return cbefore + _popcnt(eq), jnp.maximum(cfound, cnd)

    _, c = lax.fori_loop(0, nv, tie_step, (jnp.int32(0), jnp.int32(-1)))

    stage_v[...] = jnp.zeros((L,), jnp.int32) + t
    pltpu.sync_copy(stage_v, t_out.at[wid])
    stage_v[...] = jnp.zeros((L,), jnp.int32) + c
    pltpu.sync_copy(stage_v, c_out.at[wid])


def _dec_body(f_ref, t_ref, c_ref, w_ref, o_ref):
    i = pl.program_id(0)
    fb = jax.lax.bitcast_convert_type(f_ref[...], jnp.int32)
    t = t_ref[...]
    c = c_ref[...]
    idx = i * HB + jax.lax.broadcasted_iota(jnp.int32, fb.shape, 1)
    keep = (fb > t) | ((fb == t) & (idx <= c))
    vals = jnp.where(keep, f_ref[...], 0.0)
    part = jax.lax.dot_general(
        vals, w_ref[...],
        (((1,), (1,)), ((), ())),
        preferred_element_type=jnp.float32,
    )

    @pl.when(i == 0)
    def _():
        o_ref[...] = jnp.zeros_like(o_ref)

    o_ref[...] += part


def _sc_topk_call(fbits):
    mesh = plsc.VectorSubcoreMesh(core_axis_name="c", subcore_axis_name="s")
    return pl.kernel(
        _topk_tile,
        mesh=mesh,
        out_type=[
            jax.ShapeDtypeStruct((BATCH, L), jnp.int32),
            jax.ShapeDtypeStruct((BATCH, L), jnp.int32),
        ],
        scratch_types=[
            pltpu.VMEM((HIDDEN_DIM,), jnp.int32),
            pltpu.VMEM((CMAX_N,), jnp.int32),
            pltpu.VMEM((CAP,), jnp.int32),
            pltpu.VMEM((CAP,), jnp.int32),
            pltpu.VMEM((L,), jnp.int32),
        ],
        compiler_params=pltpu.CompilerParams(needs_layout_passes=False),
    )(fbits)


@jax.jit
def kernel(x, W_enc, b_enc, W_dec):
    b2d = b_enc.reshape(1, HIDDEN_DIM)

    feats = pl.pallas_call(
        _enc_body,
        grid=(N_BLK,),
        in_specs=[
            pl.BlockSpec((BATCH, INPUT_DIM), lambda i: (0, 0)),
            pl.BlockSpec((HB, INPUT_DIM), lambda i: (i, 0)),
            pl.BlockSpec((1, HB), lambda i: (0, i)),
        ],
        out_specs=pl.BlockSpec((BATCH, HB), lambda i: (0, i)),
        out_shape=jax.ShapeDtypeStruct((BATCH, HIDDEN_DIM), jnp.float32),
    )(x, W_enc, b2d)

    fbits = jax.lax.bitcast_convert_type(feats, jnp.int32)
    t16, c16 = _sc_topk_call(fbits)
    tbits, cut = t16[:, :1], c16[:, :1]

    recon = pl.pallas_call(
        _dec_body,
        grid=(N_BLK,),
        in_specs=[
            pl.BlockSpec((BATCH, HB), lambda i: (0, i)),
            pl.BlockSpec((BATCH, 1), lambda i: (0, 0)),
            pl.BlockSpec((BATCH, 1), lambda i: (0, 0)),
            pl.BlockSpec((INPUT_DIM, HB), lambda i: (0, i)),
        ],
        out_specs=pl.BlockSpec((BATCH, INPUT_DIM), lambda i: (0, 0)),
        out_shape=jax.ShapeDtypeStruct((BATCH, INPUT_DIM), jnp.float32),
    )(feats, tbits, cut, W_dec)

    return recon
